# baseline (speedup 1.0000x reference)
import functools

import jax
import jax.numpy as jnp
from jax import lax
from jax.experimental import pallas as pl
from jax.experimental.pallas import tpu as pltpu
from jax.experimental.pallas import tpu_sc as plsc

N = 10000
E = 320000
EROWS = 2560
EPAD = EROWS * 128
ACC = 10240
DUMMY = N
IN_DIM = 128
HID = 256
OUT_DIM = 128


def _sweep_block(h_ref, src_v, dst_v, rows_a, rows_b, acc, sems, npairs):
    ga, gb, sa, sb = sems
    pltpu.async_copy(h_ref.at[src_v.at[0]], rows_a, ga)

    def pair(k, carry):
        j0 = 2 * k
        pltpu.make_async_copy(h_ref.at[src_v.at[j0]], rows_a, ga).wait()
        pltpu.async_copy(rows_a, acc.at[dst_v.at[j0]], sa, add=True)

        @pl.when(k > 0)
        def _():
            pltpu.make_async_copy(rows_b, acc.at[dst_v.at[j0]], sb).wait()

        pltpu.async_copy(h_ref.at[src_v.at[j0 + 1]], rows_b, gb)
        pltpu.make_async_copy(h_ref.at[src_v.at[j0 + 1]], rows_b, gb).wait()
        pltpu.async_copy(rows_b, acc.at[dst_v.at[j0 + 1]], sb, add=True)
        pltpu.make_async_copy(rows_a, acc.at[dst_v.at[j0]], sa).wait()

        @pl.when(k < npairs - 1)
        def _():
            pltpu.async_copy(h_ref.at[src_v.at[j0 + 2]], rows_a, ga)

        return carry

    lax.fori_loop(0, npairs, pair, 0)
    pltpu.make_async_copy(rows_b, acc.at[dst_v.at[0]], sb).wait()


@functools.cache
def _sc_kernels():
    mesh = plsc.VectorSubcoreMesh(
        core_axis_name="c", subcore_axis_name="s", num_cores=2, num_subcores=16
    )

    @functools.partial(
        pl.kernel,
        out_type=jax.ShapeDtypeStruct((2 * ACC,), jnp.float32),
        mesh=mesh,
        scratch_types=[
            pltpu.VMEM((80, 128), jnp.int32),
            pltpu.VMEM((128,), jnp.float32),
            pltpu.VMEM_SHARED((ACC,), jnp.float32),
        ],
    )
    def sc_degree(dst2d, z1, dp, dst_v, ones_v, acc):
        c = lax.axis_index("c")
        s = lax.axis_index("s")
        for k in range(8):
            ones_v[pl.ds(k * 16, 16)] = jnp.ones((16,), jnp.float32)
        pltpu.sync_copy(z1.at[pl.ds(s * 640, 640)], acc.at[pl.ds(s * 640, 640)])
        pltpu.sync_copy(dst2d.at[pl.ds(c * 1280 + s * 80, 80)], dst_v)
        plsc.subcore_barrier()

        def step(j, carry):
            pltpu.sync_copy(ones_v, acc.at[dst_v.at[j]], add=True)
            return carry

        lax.fori_loop(0, 80, step, 0)
        plsc.subcore_barrier()
        pltpu.sync_copy(
            acc.at[pl.ds(s * 640, 640)], dp.at[pl.ds(c * ACC + s * 640, 640)]
        )

    @functools.partial(
        pl.kernel,
        out_type=[jax.ShapeDtypeStruct((ACC, 128), jnp.float32)] * 2,
        mesh=mesh,
        scratch_types=[
            pltpu.VMEM((40, 128), jnp.int32),
            pltpu.VMEM((40, 128), jnp.int32),
            pltpu.VMEM((128, 128), jnp.float32),
            pltpu.VMEM((128, 128), jnp.float32),
            pltpu.VMEM_SHARED((ACC, 128), jnp.float32),
            pltpu.SemaphoreType.DMA,
            pltpu.SemaphoreType.DMA,
            pltpu.SemaphoreType.DMA,
            pltpu.SemaphoreType.DMA,
        ],
    )
    def sc_agg2(
        hlo, hhi, src2d, dst2d, z2, olo, ohi,
        src_v, dst_v, rows_a, rows_b, acc, ga, gb, sa, sb
    ):
        c = lax.axis_index("c")
        s = lax.axis_index("s")
        pltpu.sync_copy(z2.at[pl.ds(s * 640, 640)], acc.at[pl.ds(s * 640, 640)])
        plsc.subcore_barrier()

        def run(h_ref):
            def outer(b, carry):
                pltpu.sync_copy(src2d.at[pl.ds(s * 160 + b * 40, 40)], src_v)
                pltpu.sync_copy(dst2d.at[pl.ds(s * 160 + b * 40, 40)], dst_v)
                _sweep_block(h_ref, src_v, dst_v, rows_a, rows_b, acc, (ga, gb, sa, sb), 20)
                return carry

            lax.fori_loop(0, 4, outer, 0)

        pl.when(c == 0)(lambda: run(hlo))
        pl.when(c == 1)(lambda: run(hhi))
        plsc.subcore_barrier()

        def wout(o_ref):
            pltpu.sync_copy(
                acc.at[pl.ds(s * 640, 640)], o_ref.at[pl.ds(s * 640, 640)]
            )

        pl.when(c == 0)(lambda: wout(olo))
        pl.when(c == 1)(lambda: wout(ohi))

    @functools.partial(
        pl.kernel,
        out_type=jax.ShapeDtypeStruct((2, ACC, 128), jnp.float32),
        mesh=mesh,
        scratch_types=[
            pltpu.VMEM((40, 128), jnp.int32),
            pltpu.VMEM((40, 128), jnp.int32),
            pltpu.VMEM((128, 128), jnp.float32),
            pltpu.VMEM((128, 128), jnp.float32),
            pltpu.VMEM_SHARED((ACC, 128), jnp.float32),
            pltpu.SemaphoreType.DMA,
            pltpu.SemaphoreType.DMA,
            pltpu.SemaphoreType.DMA,
            pltpu.SemaphoreType.DMA,
        ],
    )
    def sc_aggp(
        h, src2d, dst2d, z2, op, src_v, dst_v, rows_a, rows_b, acc, ga, gb, sa, sb
    ):
        c = lax.axis_index("c")
        s = lax.axis_index("s")
        pltpu.sync_copy(z2.at[pl.ds(s * 640, 640)], acc.at[pl.ds(s * 640, 640)])
        plsc.subcore_barrier()

        def outer(b, carry):
            base = c * 1280 + s * 80 + b * 40
            pltpu.sync_copy(src2d.at[pl.ds(base, 40)], src_v)
            pltpu.sync_copy(dst2d.at[pl.ds(base, 40)], dst_v)
            _sweep_block(h, src_v, dst_v, rows_a, rows_b, acc, (ga, gb, sa, sb), 20)
            return carry

        lax.fori_loop(0, 2, outer, 0)
        plsc.subcore_barrier()
        pltpu.sync_copy(acc.at[pl.ds(s * 640, 640)], op.at[c, pl.ds(s * 640, 640)])

    return sc_degree, sc_agg2, sc_aggp


BM = 1000
GRID = N // BM


def _k1_body(x_b, w_b, d0_b, d1_b, tlo_b, thi_b, dinv_b):
    dinv = lax.rsqrt(1.0 + d0_b[...] + d1_b[...])
    t = dinv * jnp.dot(x_b[...], w_b[...], preferred_element_type=jnp.float32)
    tlo_b[...] = t[:, :128]
    thi_b[...] = t[:, 128:]
    dinv_b[...] = dinv


_k1 = pl.pallas_call(
    _k1_body,
    grid=(GRID,),
    in_specs=[
        pl.BlockSpec((BM, IN_DIM), lambda i: (i, 0)),
        pl.BlockSpec((IN_DIM, HID), lambda i: (0, 0)),
        pl.BlockSpec((BM, 1), lambda i: (i, 0)),
        pl.BlockSpec((BM, 1), lambda i: (i, 0)),
    ],
    out_specs=[
        pl.BlockSpec((BM, 128), lambda i: (i, 0)),
        pl.BlockSpec((BM, 128), lambda i: (i, 0)),
        pl.BlockSpec((BM, 1), lambda i: (i, 0)),
    ],
    out_shape=[
        jax.ShapeDtypeStruct((N, 128), jnp.float32),
        jax.ShapeDtypeStruct((N, 128), jnp.float32),
        jax.ShapeDtypeStruct((N, 1), jnp.float32),
    ],
)


def _k2_body(slo, shi, tlo, thi, dinv_b, b_b, w_b, olo, ohi):
    dinv = dinv_b[...]
    agg = jnp.concatenate([slo[...] + tlo[...], shi[...] + thi[...]], axis=1)
    u = jnp.maximum(dinv * agg + b_b[...], 0.0)
    o = dinv * jnp.dot(u, w_b[...], preferred_element_type=jnp.float32)
    olo[...] = o[:, :128]
    ohi[...] = o[:, 128:]


_k2 = pl.pallas_call(
    _k2_body,
    grid=(GRID,),
    in_specs=[
        pl.BlockSpec((BM, 128), lambda i: (i, 0)),
        pl.BlockSpec((BM, 128), lambda i: (i, 0)),
        pl.BlockSpec((BM, 128), lambda i: (i, 0)),
        pl.BlockSpec((BM, 128), lambda i: (i, 0)),
        pl.BlockSpec((BM, 1), lambda i: (i, 0)),
        pl.BlockSpec((1, HID), lambda i: (0, 0)),
        pl.BlockSpec((HID, HID), lambda i: (0, 0)),
    ],
    out_specs=[
        pl.BlockSpec((BM, 128), lambda i: (i, 0)),
        pl.BlockSpec((BM, 128), lambda i: (i, 0)),
    ],
    out_shape=[
        jax.ShapeDtypeStruct((N, 128), jnp.float32),
        jax.ShapeDtypeStruct((N, 128), jnp.float32),
    ],
)


def _k3_body(slo, shi, tlo, thi, dinv_b, b_b, w_b, o_b):
    dinv = dinv_b[...]
    agg = jnp.concatenate([slo[...] + tlo[...], shi[...] + thi[...]], axis=1)
    u = jnp.maximum(dinv * agg + b_b[...], 0.0)
    o_b[...] = dinv * jnp.dot(u, w_b[...], preferred_element_type=jnp.float32)


_k3 = pl.pallas_call(
    _k3_body,
    grid=(GRID,),
    in_specs=[
        pl.BlockSpec((BM, 128), lambda i: (i, 0)),
        pl.BlockSpec((BM, 128), lambda i: (i, 0)),
        pl.BlockSpec((BM, 128), lambda i: (i, 0)),
        pl.BlockSpec((BM, 128), lambda i: (i, 0)),
        pl.BlockSpec((BM, 1), lambda i: (i, 0)),
        pl.BlockSpec((1, HID), lambda i: (0, 0)),
        pl.BlockSpec((HID, OUT_DIM), lambda i: (0, 0)),
    ],
    out_specs=pl.BlockSpec((BM, OUT_DIM), lambda i: (i, 0)),
    out_shape=jax.ShapeDtypeStruct((N, OUT_DIM), jnp.float32),
)


def _k4_body(p0, p1, t3, dinv_b, b_b, o_b):
    o_b[...] = dinv_b[...] * (p0[...] + p1[...] + t3[...]) + b_b[...]


_k4 = pl.pallas_call(
    _k4_body,
    grid=(GRID,),
    in_specs=[
        pl.BlockSpec((BM, OUT_DIM), lambda i: (i, 0)),
        pl.BlockSpec((BM, OUT_DIM), lambda i: (i, 0)),
        pl.BlockSpec((BM, OUT_DIM), lambda i: (i, 0)),
        pl.BlockSpec((BM, 1), lambda i: (i, 0)),
        pl.BlockSpec((1, OUT_DIM), lambda i: (0, 0)),
    ],
    out_specs=pl.BlockSpec((BM, OUT_DIM), lambda i: (i, 0)),
    out_shape=jax.ShapeDtypeStruct((N, OUT_DIM), jnp.float32),
)


def kernel(x, edge_index, W1, b1, W2, b2, W3, b3):
    sc_degree, sc_agg2, sc_aggp = _sc_kernels()
    ei = edge_index.astype(jnp.int32)
    pad = EPAD - E
    src2d = jnp.concatenate([ei[0], jnp.zeros((pad,), jnp.int32)]).reshape(EROWS, 128)
    pad_dst = DUMMY + jnp.arange(pad, dtype=jnp.int32) % (ACC - N)
    dst2d = jnp.concatenate([ei[1], pad_dst]).reshape(EROWS, 128)
    z1 = jnp.zeros((ACC,), jnp.float32)
    z2 = jnp.zeros((ACC, 128), jnp.float32)

    dp = sc_degree(dst2d, z1)
    d0 = dp[:N].reshape(N, 1)
    d1 = dp[ACC:ACC + N].reshape(N, 1)

    t1lo, t1hi, dinv = _k1(x, W1, d0, d1)
    s1lo, s1hi = sc_agg2(t1lo, t1hi, src2d, dst2d, z2)
    t2lo, t2hi = _k2(s1lo, s1hi, t1lo, t1hi, dinv, b1.reshape(1, HID), W2)
    s2lo, s2hi = sc_agg2(t2lo, t2hi, src2d, dst2d, z2)
    t3 = _k3(s2lo, s2hi, t2lo, t2hi, dinv, b2.reshape(1, HID), W3)
    p = sc_aggp(t3, src2d, dst2d, z2)
    out = _k4(p[0], p[1], t3, dinv, b3.reshape(1, OUT_DIM))
    return out


# speedup vs baseline: 1.0195x; 1.0195x over previous
import functools

import jax
import jax.numpy as jnp
from jax import lax
from jax.experimental import pallas as pl
from jax.experimental.pallas import tpu as pltpu
from jax.experimental.pallas import tpu_sc as plsc

N = 10000
E = 320000
EROWS = 2560
EPAD = EROWS * 128
ACC = 10240
DUMMY = N
IN_DIM = 128
HID = 256
OUT_DIM = 128


def _sweep_block(h_ref, src_v, dst_v, rows_a, rows_b, acc, sems, npairs):
    ga, gb, sa, sb = sems
    pltpu.async_copy(h_ref.at[src_v.at[0]], rows_a, ga)

    def pair(k, carry):
        j0 = 2 * k
        pltpu.make_async_copy(h_ref.at[src_v.at[j0]], rows_a, ga).wait()
        pltpu.async_copy(rows_a, acc.at[dst_v.at[j0]], sa, add=True)

        @pl.when(k > 0)
        def _():
            pltpu.make_async_copy(rows_b, acc.at[dst_v.at[j0]], sb).wait()

        pltpu.async_copy(h_ref.at[src_v.at[j0 + 1]], rows_b, gb)
        pltpu.make_async_copy(h_ref.at[src_v.at[j0 + 1]], rows_b, gb).wait()
        pltpu.async_copy(rows_b, acc.at[dst_v.at[j0 + 1]], sb, add=True)
        pltpu.make_async_copy(rows_a, acc.at[dst_v.at[j0]], sa).wait()

        @pl.when(k < npairs - 1)
        def _():
            pltpu.async_copy(h_ref.at[src_v.at[j0 + 2]], rows_a, ga)

        return carry

    lax.fori_loop(0, npairs, pair, 0)
    pltpu.make_async_copy(rows_b, acc.at[dst_v.at[0]], sb).wait()


@functools.cache
def _sc_kernels():
    mesh = plsc.VectorSubcoreMesh(
        core_axis_name="c", subcore_axis_name="s", num_cores=2, num_subcores=16
    )

    @functools.partial(
        pl.kernel,
        out_type=jax.ShapeDtypeStruct((2 * ACC,), jnp.float32),
        mesh=mesh,
        scratch_types=[
            pltpu.VMEM((80, 128), jnp.int32),
            pltpu.VMEM((128,), jnp.float32),
            pltpu.VMEM_SHARED((ACC,), jnp.float32),
        ],
    )
    def sc_degree(dst2d, z1, dp, dst_v, ones_v, acc):
        c = lax.axis_index("c")
        s = lax.axis_index("s")
        for k in range(8):
            ones_v[pl.ds(k * 16, 16)] = jnp.ones((16,), jnp.float32)
        pltpu.sync_copy(z1.at[pl.ds(s * 640, 640)], acc.at[pl.ds(s * 640, 640)])
        pltpu.sync_copy(dst2d.at[pl.ds(c * 1280 + s * 80, 80)], dst_v)
        plsc.subcore_barrier()

        def step(j, carry):
            pltpu.sync_copy(ones_v, acc.at[dst_v.at[j]], add=True)
            return carry

        lax.fori_loop(0, 80, step, 0)
        plsc.subcore_barrier()
        pltpu.sync_copy(
            acc.at[pl.ds(s * 640, 640)], dp.at[pl.ds(c * ACC + s * 640, 640)]
        )

    @functools.partial(
        pl.kernel,
        out_type=[jax.ShapeDtypeStruct((ACC, 128), jnp.float32)] * 2,
        mesh=mesh,
        scratch_types=[
            pltpu.VMEM((40, 128), jnp.int32),
            pltpu.VMEM((40, 128), jnp.int32),
            pltpu.VMEM((128, 128), jnp.float32),
            pltpu.VMEM((128, 128), jnp.float32),
            pltpu.VMEM_SHARED((ACC, 128), jnp.float32),
            pltpu.SemaphoreType.DMA,
            pltpu.SemaphoreType.DMA,
            pltpu.SemaphoreType.DMA,
            pltpu.SemaphoreType.DMA,
        ],
    )
    def sc_agg2(
        hlo, hhi, src2d, dst2d, z2, olo, ohi,
        src_v, dst_v, rows_a, rows_b, acc, ga, gb, sa, sb
    ):
        c = lax.axis_index("c")
        s = lax.axis_index("s")
        pltpu.sync_copy(z2.at[pl.ds(s * 640, 640)], acc.at[pl.ds(s * 640, 640)])
        plsc.subcore_barrier()

        def run(h_ref):
            def outer(b, carry):
                pltpu.sync_copy(src2d.at[pl.ds(s * 160 + b * 40, 40)], src_v)
                pltpu.sync_copy(dst2d.at[pl.ds(s * 160 + b * 40, 40)], dst_v)
                _sweep_block(h_ref, src_v, dst_v, rows_a, rows_b, acc, (ga, gb, sa, sb), 20)
                return carry

            lax.fori_loop(0, 4, outer, 0)

        pl.when(c == 0)(lambda: run(hlo))
        pl.when(c == 1)(lambda: run(hhi))
        plsc.subcore_barrier()

        def wout(o_ref):
            pltpu.sync_copy(
                acc.at[pl.ds(s * 640, 640)], o_ref.at[pl.ds(s * 640, 640)]
            )

        pl.when(c == 0)(lambda: wout(olo))
        pl.when(c == 1)(lambda: wout(ohi))

    @functools.partial(
        pl.kernel,
        out_type=jax.ShapeDtypeStruct((2, ACC, 128), jnp.float32),
        mesh=mesh,
        scratch_types=[
            pltpu.VMEM((40, 128), jnp.int32),
            pltpu.VMEM((40, 128), jnp.int32),
            pltpu.VMEM((128, 128), jnp.float32),
            pltpu.VMEM((128, 128), jnp.float32),
            pltpu.VMEM_SHARED((ACC, 128), jnp.float32),
            pltpu.SemaphoreType.DMA,
            pltpu.SemaphoreType.DMA,
            pltpu.SemaphoreType.DMA,
            pltpu.SemaphoreType.DMA,
        ],
    )
    def sc_aggp(
        h, h2, src2d, dst2d, z2, op, src_v, dst_v, rows_a, rows_b, acc, ga, gb, sa, sb
    ):
        c = lax.axis_index("c")
        s = lax.axis_index("s")
        pltpu.sync_copy(z2.at[pl.ds(s * 640, 640)], acc.at[pl.ds(s * 640, 640)])
        plsc.subcore_barrier()

        def run(h_ref, base0):
            def outer(b, carry):
                base = base0 + s * 80 + b * 40
                pltpu.sync_copy(src2d.at[pl.ds(base, 40)], src_v)
                pltpu.sync_copy(dst2d.at[pl.ds(base, 40)], dst_v)
                _sweep_block(
                    h_ref, src_v, dst_v, rows_a, rows_b, acc, (ga, gb, sa, sb), 20
                )
                return carry

            lax.fori_loop(0, 2, outer, 0)

        pl.when(c == 0)(lambda: run(h, 0))
        pl.when(c == 1)(lambda: run(h2, 1280))
        plsc.subcore_barrier()
        pltpu.sync_copy(acc.at[pl.ds(s * 640, 640)], op.at[c, pl.ds(s * 640, 640)])

    return sc_degree, sc_agg2, sc_aggp


BM = 1000
GRID = N // BM


def _k1_body(x_b, w_b, d0_b, d1_b, tlo_b, thi_b, dinv_b):
    dinv = lax.rsqrt(1.0 + d0_b[...] + d1_b[...])
    t = dinv * jnp.dot(x_b[...], w_b[...], preferred_element_type=jnp.float32)
    tlo_b[...] = t[:, :128]
    thi_b[...] = t[:, 128:]
    dinv_b[...] = dinv


_k1 = pl.pallas_call(
    _k1_body,
    grid=(GRID,),
    in_specs=[
        pl.BlockSpec((BM, IN_DIM), lambda i: (i, 0)),
        pl.BlockSpec((IN_DIM, HID), lambda i: (0, 0)),
        pl.BlockSpec((BM, 1), lambda i: (i, 0)),
        pl.BlockSpec((BM, 1), lambda i: (i, 0)),
    ],
    out_specs=[
        pl.BlockSpec((BM, 128), lambda i: (i, 0)),
        pl.BlockSpec((BM, 128), lambda i: (i, 0)),
        pl.BlockSpec((BM, 1), lambda i: (i, 0)),
    ],
    out_shape=[
        jax.ShapeDtypeStruct((N, 128), jnp.float32),
        jax.ShapeDtypeStruct((N, 128), jnp.float32),
        jax.ShapeDtypeStruct((N, 1), jnp.float32),
    ],
)


def _k2_body(slo, shi, tlo, thi, dinv_b, b_b, w_b, olo, ohi):
    dinv = dinv_b[...]
    agg = jnp.concatenate([slo[...] + tlo[...], shi[...] + thi[...]], axis=1)
    u = jnp.maximum(dinv * agg + b_b[...], 0.0)
    o = dinv * jnp.dot(u, w_b[...], preferred_element_type=jnp.float32)
    olo[...] = o[:, :128]
    ohi[...] = o[:, 128:]


_k2 = pl.pallas_call(
    _k2_body,
    grid=(GRID,),
    in_specs=[
        pl.BlockSpec((BM, 128), lambda i: (i, 0)),
        pl.BlockSpec((BM, 128), lambda i: (i, 0)),
        pl.BlockSpec((BM, 128), lambda i: (i, 0)),
        pl.BlockSpec((BM, 128), lambda i: (i, 0)),
        pl.BlockSpec((BM, 1), lambda i: (i, 0)),
        pl.BlockSpec((1, HID), lambda i: (0, 0)),
        pl.BlockSpec((HID, HID), lambda i: (0, 0)),
    ],
    out_specs=[
        pl.BlockSpec((BM, 128), lambda i: (i, 0)),
        pl.BlockSpec((BM, 128), lambda i: (i, 0)),
    ],
    out_shape=[
        jax.ShapeDtypeStruct((N, 128), jnp.float32),
        jax.ShapeDtypeStruct((N, 128), jnp.float32),
    ],
)


def _k3_body(slo, shi, tlo, thi, dinv_b, b_b, w_b, o_b, o2_b):
    dinv = dinv_b[...]
    agg = jnp.concatenate([slo[...] + tlo[...], shi[...] + thi[...]], axis=1)
    u = jnp.maximum(dinv * agg + b_b[...], 0.0)
    o = dinv * jnp.dot(u, w_b[...], preferred_element_type=jnp.float32)
    o_b[...] = o
    o2_b[...] = o


_k3 = pl.pallas_call(
    _k3_body,
    grid=(GRID,),
    in_specs=[
        pl.BlockSpec((BM, 128), lambda i: (i, 0)),
        pl.BlockSpec((BM, 128), lambda i: (i, 0)),
        pl.BlockSpec((BM, 128), lambda i: (i, 0)),
        pl.BlockSpec((BM, 128), lambda i: (i, 0)),
        pl.BlockSpec((BM, 1), lambda i: (i, 0)),
        pl.BlockSpec((1, HID), lambda i: (0, 0)),
        pl.BlockSpec((HID, OUT_DIM), lambda i: (0, 0)),
    ],
    out_specs=[pl.BlockSpec((BM, OUT_DIM), lambda i: (i, 0))] * 2,
    out_shape=[jax.ShapeDtypeStruct((N, OUT_DIM), jnp.float32)] * 2,
)


def _k4_body(p0, p1, t3, dinv_b, b_b, o_b):
    o_b[...] = dinv_b[...] * (p0[...] + p1[...] + t3[...]) + b_b[...]


_k4 = pl.pallas_call(
    _k4_body,
    grid=(GRID,),
    in_specs=[
        pl.BlockSpec((BM, OUT_DIM), lambda i: (i, 0)),
        pl.BlockSpec((BM, OUT_DIM), lambda i: (i, 0)),
        pl.BlockSpec((BM, OUT_DIM), lambda i: (i, 0)),
        pl.BlockSpec((BM, 1), lambda i: (i, 0)),
        pl.BlockSpec((1, OUT_DIM), lambda i: (0, 0)),
    ],
    out_specs=pl.BlockSpec((BM, OUT_DIM), lambda i: (i, 0)),
    out_shape=jax.ShapeDtypeStruct((N, OUT_DIM), jnp.float32),
)


def kernel(x, edge_index, W1, b1, W2, b2, W3, b3):
    sc_degree, sc_agg2, sc_aggp = _sc_kernels()
    ei = edge_index.astype(jnp.int32)
    pad = EPAD - E
    src2d = jnp.concatenate([ei[0], jnp.zeros((pad,), jnp.int32)]).reshape(EROWS, 128)
    pad_dst = DUMMY + jnp.arange(pad, dtype=jnp.int32) % (ACC - N)
    dst2d = jnp.concatenate([ei[1], pad_dst]).reshape(EROWS, 128)
    z1 = jnp.zeros((ACC,), jnp.float32)
    z2 = jnp.zeros((ACC, 128), jnp.float32)

    dp = sc_degree(dst2d, z1)
    d0 = dp[:N].reshape(N, 1)
    d1 = dp[ACC:ACC + N].reshape(N, 1)

    t1lo, t1hi, dinv = _k1(x, W1, d0, d1)
    s1lo, s1hi = sc_agg2(t1lo, t1hi, src2d, dst2d, z2)
    t2lo, t2hi = _k2(s1lo, s1hi, t1lo, t1hi, dinv, b1.reshape(1, HID), W2)
    s2lo, s2hi = sc_agg2(t2lo, t2hi, src2d, dst2d, z2)
    t3, t3b = _k3(s2lo, s2hi, t2lo, t2hi, dinv, b2.reshape(1, HID), W3)
    p = sc_aggp(t3, t3b, src2d, dst2d, z2)
    out = _k4(p[0], p[1], t3, dinv, b3.reshape(1, OUT_DIM))
    return out


# speedup vs baseline: 1.0920x; 1.0711x over previous
import functools

import jax
import jax.numpy as jnp
from jax import lax
from jax.experimental import pallas as pl
from jax.experimental.pallas import tpu as pltpu
from jax.experimental.pallas import tpu_sc as plsc

N = 10000
E = 320000
EROWS = 2560
EPAD = EROWS * 128
ACC = 10240
DUMMY = N
IN_DIM = 128
HID = 256
OUT_DIM = 128


def _sweep_block(h_ref, src_v, dst_v, rows_a, rows_b, acc, sems, npairs):
    ga, gb, sa, sb = sems
    pltpu.async_copy(h_ref.at[src_v.at[0]], rows_a, ga)

    def pair(k, carry):
        j0 = 2 * k
        pltpu.make_async_copy(h_ref.at[src_v.at[j0]], rows_a, ga).wait()
        pltpu.async_copy(rows_a, acc.at[dst_v.at[j0]], sa, add=True)

        @pl.when(k > 0)
        def _():
            pltpu.make_async_copy(rows_b, acc.at[dst_v.at[j0]], sb).wait()

        pltpu.async_copy(h_ref.at[src_v.at[j0 + 1]], rows_b, gb)
        pltpu.make_async_copy(h_ref.at[src_v.at[j0 + 1]], rows_b, gb).wait()
        pltpu.async_copy(rows_b, acc.at[dst_v.at[j0 + 1]], sb, add=True)
        pltpu.make_async_copy(rows_a, acc.at[dst_v.at[j0]], sa).wait()

        @pl.when(k < npairs - 1)
        def _():
            pltpu.async_copy(h_ref.at[src_v.at[j0 + 2]], rows_a, ga)

        return carry

    lax.fori_loop(0, npairs, pair, 0)
    pltpu.make_async_copy(rows_b, acc.at[dst_v.at[0]], sb).wait()


@functools.cache
def _sc_kernels():
    mesh = plsc.VectorSubcoreMesh(
        core_axis_name="c", subcore_axis_name="s", num_cores=2, num_subcores=16
    )

    @functools.partial(
        pl.kernel,
        out_type=jax.ShapeDtypeStruct((2 * ACC,), jnp.float32),
        mesh=mesh,
        scratch_types=[
            pltpu.VMEM((80, 128), jnp.int32),
            pltpu.VMEM((128,), jnp.float32),
            pltpu.VMEM_SHARED((ACC,), jnp.float32),
        ],
    )
    def sc_degree(dst2d, z1, dp, dst_v, ones_v, acc):
        c = lax.axis_index("c")
        s = lax.axis_index("s")
        for k in range(8):
            ones_v[pl.ds(k * 16, 16)] = jnp.ones((16,), jnp.float32)
        pltpu.sync_copy(z1.at[pl.ds(s * 640, 640)], acc.at[pl.ds(s * 640, 640)])
        pltpu.sync_copy(dst2d.at[pl.ds(c * 1280 + s * 80, 80)], dst_v)
        plsc.subcore_barrier()

        def step(j, carry):
            pltpu.sync_copy(ones_v, acc.at[dst_v.at[j]], add=True)
            return carry

        lax.fori_loop(0, 80, step, 0)
        plsc.subcore_barrier()
        pltpu.sync_copy(
            acc.at[pl.ds(s * 640, 640)], dp.at[pl.ds(c * ACC + s * 640, 640)]
        )

    @functools.partial(
        pl.kernel,
        out_type=[jax.ShapeDtypeStruct((ACC, 128), jnp.float32)] * 2,
        mesh=mesh,
        scratch_types=[
            pltpu.VMEM((40, 128), jnp.int32),
            pltpu.VMEM((40, 128), jnp.int32),
            pltpu.VMEM((128, 128), jnp.float32),
            pltpu.VMEM((128, 128), jnp.float32),
            pltpu.VMEM_SHARED((ACC, 128), jnp.float32),
            pltpu.SemaphoreType.DMA,
            pltpu.SemaphoreType.DMA,
            pltpu.SemaphoreType.DMA,
            pltpu.SemaphoreType.DMA,
        ],
    )
    def sc_agg2(
        hlo, hhi, src2d, dst2d, z2, olo, ohi,
        src_v, dst_v, rows_a, rows_b, acc, ga, gb, sa, sb
    ):
        c = lax.axis_index("c")
        s = lax.axis_index("s")
        pltpu.sync_copy(z2.at[pl.ds(s * 640, 640)], acc.at[pl.ds(s * 640, 640)])
        plsc.subcore_barrier()

        def run(h_ref):
            def outer(b, carry):
                pltpu.sync_copy(src2d.at[pl.ds(s * 160 + b * 40, 40)], src_v)
                pltpu.sync_copy(dst2d.at[pl.ds(s * 160 + b * 40, 40)], dst_v)
                _sweep_block(h_ref, src_v, dst_v, rows_a, rows_b, acc, (ga, gb, sa, sb), 20)
                return carry

            lax.fori_loop(0, 4, outer, 0)

        pl.when(c == 0)(lambda: run(hlo))
        pl.when(c == 1)(lambda: run(hhi))
        plsc.subcore_barrier()

        def wout(o_ref):
            pltpu.sync_copy(
                acc.at[pl.ds(s * 640, 640)], o_ref.at[pl.ds(s * 640, 640)]
            )

        pl.when(c == 0)(lambda: wout(olo))
        pl.when(c == 1)(lambda: wout(ohi))

    @functools.partial(
        pl.kernel,
        out_type=jax.ShapeDtypeStruct((2, ACC, 128), jnp.float32),
        mesh=mesh,
        scratch_types=[
            pltpu.VMEM((40, 128), jnp.int32),
            pltpu.VMEM((40, 128), jnp.int32),
            pltpu.VMEM((128, 128), jnp.float32),
            pltpu.VMEM((128, 128), jnp.float32),
            pltpu.VMEM_SHARED((ACC, 128), jnp.float32),
            pltpu.SemaphoreType.DMA,
            pltpu.SemaphoreType.DMA,
            pltpu.SemaphoreType.DMA,
            pltpu.SemaphoreType.DMA,
        ],
    )
    def sc_aggp(
        h, h2, src2d, dst2d, z2, op, src_v, dst_v, rows_a, rows_b, acc, ga, gb, sa, sb
    ):
        c = lax.axis_index("c")
        s = lax.axis_index("s")
        pltpu.sync_copy(z2.at[pl.ds(s * 640, 640)], acc.at[pl.ds(s * 640, 640)])
        plsc.subcore_barrier()

        def run(h_ref, base0):
            def outer(b, carry):
                base = s * 160 + b * 80 + base0
                pltpu.sync_copy(src2d.at[pl.ds(base, 40)], src_v)
                pltpu.sync_copy(dst2d.at[pl.ds(base, 40)], dst_v)
                _sweep_block(
                    h_ref, src_v, dst_v, rows_a, rows_b, acc, (ga, gb, sa, sb), 20
                )
                return carry

            lax.fori_loop(0, 2, outer, 0)

        pl.when(c == 0)(lambda: run(h, 0))
        pl.when(c == 1)(lambda: run(h2, 40))
        plsc.subcore_barrier()
        pltpu.sync_copy(acc.at[pl.ds(s * 640, 640)], op.at[c, pl.ds(s * 640, 640)])

    return sc_degree, sc_agg2, sc_aggp


BM = 1000
GRID = N // BM


def _k1_body(x_b, w_b, d0_b, d1_b, tlo_b, thi_b, dinv_b):
    dinv = lax.rsqrt(1.0 + d0_b[...] + d1_b[...])
    t = dinv * jnp.dot(x_b[...], w_b[...], preferred_element_type=jnp.float32)
    tlo_b[...] = t[:, :128]
    thi_b[...] = t[:, 128:]
    dinv_b[...] = dinv


_k1 = pl.pallas_call(
    _k1_body,
    grid=(GRID,),
    in_specs=[
        pl.BlockSpec((BM, IN_DIM), lambda i: (i, 0)),
        pl.BlockSpec((IN_DIM, HID), lambda i: (0, 0)),
        pl.BlockSpec((BM, 1), lambda i: (i, 0)),
        pl.BlockSpec((BM, 1), lambda i: (i, 0)),
    ],
    out_specs=[
        pl.BlockSpec((BM, 128), lambda i: (i, 0)),
        pl.BlockSpec((BM, 128), lambda i: (i, 0)),
        pl.BlockSpec((BM, 1), lambda i: (i, 0)),
    ],
    out_shape=[
        jax.ShapeDtypeStruct((N, 128), jnp.float32),
        jax.ShapeDtypeStruct((N, 128), jnp.float32),
        jax.ShapeDtypeStruct((N, 1), jnp.float32),
    ],
)


def _k2_body(slo, shi, tlo, thi, dinv_b, b_b, w_b, olo, ohi):
    dinv = dinv_b[...]
    agg = jnp.concatenate([slo[...] + tlo[...], shi[...] + thi[...]], axis=1)
    u = jnp.maximum(dinv * agg + b_b[...], 0.0)
    o = dinv * jnp.dot(u, w_b[...], preferred_element_type=jnp.float32)
    olo[...] = o[:, :128]
    ohi[...] = o[:, 128:]


_k2 = pl.pallas_call(
    _k2_body,
    grid=(GRID,),
    in_specs=[
        pl.BlockSpec((BM, 128), lambda i: (i, 0)),
        pl.BlockSpec((BM, 128), lambda i: (i, 0)),
        pl.BlockSpec((BM, 128), lambda i: (i, 0)),
        pl.BlockSpec((BM, 128), lambda i: (i, 0)),
        pl.BlockSpec((BM, 1), lambda i: (i, 0)),
        pl.BlockSpec((1, HID), lambda i: (0, 0)),
        pl.BlockSpec((HID, HID), lambda i: (0, 0)),
    ],
    out_specs=[
        pl.BlockSpec((BM, 128), lambda i: (i, 0)),
        pl.BlockSpec((BM, 128), lambda i: (i, 0)),
    ],
    out_shape=[
        jax.ShapeDtypeStruct((N, 128), jnp.float32),
        jax.ShapeDtypeStruct((N, 128), jnp.float32),
    ],
)


def _k3_body(slo, shi, tlo, thi, dinv_b, b_b, w_b, o_b, o2_b):
    dinv = dinv_b[...]
    agg = jnp.concatenate([slo[...] + tlo[...], shi[...] + thi[...]], axis=1)
    u = jnp.maximum(dinv * agg + b_b[...], 0.0)
    o = dinv * jnp.dot(u, w_b[...], preferred_element_type=jnp.float32)
    o_b[...] = o
    o2_b[...] = o


_k3 = pl.pallas_call(
    _k3_body,
    grid=(GRID,),
    in_specs=[
        pl.BlockSpec((BM, 128), lambda i: (i, 0)),
        pl.BlockSpec((BM, 128), lambda i: (i, 0)),
        pl.BlockSpec((BM, 128), lambda i: (i, 0)),
        pl.BlockSpec((BM, 128), lambda i: (i, 0)),
        pl.BlockSpec((BM, 1), lambda i: (i, 0)),
        pl.BlockSpec((1, HID), lambda i: (0, 0)),
        pl.BlockSpec((HID, OUT_DIM), lambda i: (0, 0)),
    ],
    out_specs=[pl.BlockSpec((BM, OUT_DIM), lambda i: (i, 0))] * 2,
    out_shape=[jax.ShapeDtypeStruct((N, OUT_DIM), jnp.float32)] * 2,
)


def _k4_body(p0, p1, t3, dinv_b, b_b, o_b):
    o_b[...] = dinv_b[...] * (p0[...] + p1[...] + t3[...]) + b_b[...]


_k4 = pl.pallas_call(
    _k4_body,
    grid=(GRID,),
    in_specs=[
        pl.BlockSpec((BM, OUT_DIM), lambda i: (i, 0)),
        pl.BlockSpec((BM, OUT_DIM), lambda i: (i, 0)),
        pl.BlockSpec((BM, OUT_DIM), lambda i: (i, 0)),
        pl.BlockSpec((BM, 1), lambda i: (i, 0)),
        pl.BlockSpec((1, OUT_DIM), lambda i: (0, 0)),
    ],
    out_specs=pl.BlockSpec((BM, OUT_DIM), lambda i: (i, 0)),
    out_shape=jax.ShapeDtypeStruct((N, OUT_DIM), jnp.float32),
)


def kernel(x, edge_index, W1, b1, W2, b2, W3, b3):
    sc_degree, sc_agg2, sc_aggp = _sc_kernels()
    ei = edge_index.astype(jnp.int32)
    pad = EPAD - E
    src2d = jnp.concatenate([ei[0], jnp.zeros((pad,), jnp.int32)]).reshape(EROWS, 128)
    pad_dst = DUMMY + jnp.arange(pad, dtype=jnp.int32) % (ACC - N)
    dst2d = jnp.concatenate([ei[1], pad_dst]).reshape(EROWS, 128)
    z1 = jnp.zeros((ACC,), jnp.float32)
    z2 = jnp.zeros((ACC, 128), jnp.float32)

    dp = sc_degree(dst2d, z1)
    d0 = dp[:N].reshape(N, 1)
    d1 = dp[ACC:ACC + N].reshape(N, 1)

    t1lo, t1hi, dinv = _k1(x, W1, d0, d1)
    s1lo, s1hi = sc_agg2(t1lo, t1hi, src2d, dst2d, z2)
    t2lo, t2hi = _k2(s1lo, s1hi, t1lo, t1hi, dinv, b1.reshape(1, HID), W2)
    s2lo, s2hi = sc_agg2(t2lo, t2hi, src2d, dst2d, z2)
    t3, t3b = _k3(s2lo, s2hi, t2lo, t2hi, dinv, b2.reshape(1, HID), W3)
    p = sc_aggp(t3, t3b, src2d, dst2d, z2)
    out = _k4(p[0], p[1], t3, dinv, b3.reshape(1, OUT_DIM))
    return out
